# baseline (speedup 1.0000x reference)
import functools

import jax
import jax.numpy as jnp
from jax import lax
from jax.experimental import pallas as pl
from jax.experimental.pallas import tpu as pltpu
from jax.experimental.pallas import tpu_sc as plsc

_B = 16384
_IN_DIM = 768
_H1 = 512
_H2 = 256
_E_DIM = 64
_L = 3
_K = 256
_NE = 10
_BETA = 0.001
_BM = 512
_NB = _B // _BM

_SC_CORES = 2
_SC_SUBCORES = 16
_NW = _SC_CORES * _SC_SUBCORES
_BW = _B // _NW
_QW = 2 * _E_DIM
_CH = 128
_NCH = _BW // _CH


def _row_spec(w):
    return pl.BlockSpec((_BM, w), lambda i: (i, 0))


def _full_spec(shape):
    nd = len(shape)
    return pl.BlockSpec(shape, lambda i: (0,) * nd)


def _st_update(r, xhat, q):
    u = r + (q - r)
    return r - u, xhat + u


def _enc_body(x_ref, We0, be0, We1, be1, We2, be2, z_ref):
    h = jnp.maximum(jnp.dot(x_ref[...], We0[...],
                            preferred_element_type=jnp.float32) + be0[...], 0.0)
    h = jnp.maximum(jnp.dot(h, We1[...],
                            preferred_element_type=jnp.float32) + be1[...], 0.0)
    z_ref[...] = jnp.dot(h, We2[...],
                         preferred_element_type=jnp.float32) + be2[...]


def _vq_body(*refs):
    (z_ref, lab_ref, rn_ref, cn_ref, cb_ref), rest = refs[:5], refs[5:]
    xq_refs, fi_ref = rest[:-1], rest[-1]
    r = z_ref[...]
    xhat = jnp.zeros_like(r)
    for q_ref in xq_refs:
        r, xhat = _st_update(r, xhat, q_ref[...][:, :_E_DIM])
    sc = jnp.dot(r, cb_ref[...].T, preferred_element_type=jnp.float32)
    d = (rn_ref[...] - 2.0 * sc) + cn_ref[...]
    col = jax.lax.broadcasted_iota(jnp.int32, (1, _NE * _K), 1)
    d = jnp.where((col // _K) == lab_ref[...], d, jnp.float32(1e30))
    fi_ref[...] = jnp.argmin(d, axis=1).astype(jnp.int32)[:, None]


def _dec_body(z_ref, xq0_ref, xq1_ref, xq2_ref,
              Wd0, bd0, Wd1, bd1, Wd2, bd2,
              out_ref, xhat_ref, l0_ref, l1_ref, l2_ref):
    i = pl.program_id(0)
    r = z_ref[...]
    xhat = jnp.zeros_like(r)
    loss_refs = (l0_ref, l1_ref, l2_ref)
    for l, q_ref in enumerate((xq0_ref, xq1_ref, xq2_ref)):
        q = q_ref[...][:, :_E_DIM]
        diff = q - r
        lsum = jnp.sum(diff * diff).reshape(1, 1)
        r, xhat = _st_update(r, xhat, q)

        @pl.when(i == 0)
        def _():
            loss_refs[l][...] = lsum

        @pl.when(i != 0)
        def _():
            loss_refs[l][...] = loss_refs[l][...] + lsum

    xhat_ref[...] = xhat
    h = jnp.maximum(jnp.dot(xhat, Wd0[...],
                            preferred_element_type=jnp.float32) + bd0[...], 0.0)
    h = jnp.maximum(jnp.dot(h, Wd1[...],
                            preferred_element_type=jnp.float32) + bd1[...], 0.0)
    out_ref[...] = jnp.dot(h, Wd2[...],
                           preferred_element_type=jnp.float32) + bd2[...]


def _sc_gather(table, idx2d):
    mesh = plsc.VectorSubcoreMesh(core_axis_name="c", subcore_axis_name="s")

    @functools.partial(
        pl.kernel, mesh=mesh,
        out_type=jax.ShapeDtypeStruct((_B, _QW), jnp.float32),
        scratch_types=[
            pltpu.VMEM((_NCH, _CH), jnp.int32),
            pltpu.VMEM((_BW, _QW), jnp.float32),
            pltpu.SemaphoreType.DMA,
        ],
    )
    def k(table_hbm, idx_hbm, out_hbm, idx_v, rows_v, sem):
        wid = lax.axis_index("s") * _SC_CORES + lax.axis_index("c")
        pltpu.sync_copy(idx_hbm.at[pl.ds(wid * _NCH, _NCH)], idx_v)
        copies = [
            pltpu.async_copy(table_hbm.at[idx_v.at[j]],
                             rows_v.at[pl.ds(j * _CH, _CH)], sem)
            for j in range(_NCH)
        ]
        for c in copies:
            c.wait()
        pltpu.sync_copy(rows_v, out_hbm.at[pl.ds(wid * _BW, _BW)])

    return k(table, idx2d)


def _vq_call(z, lab2, rn, cn, cbl, xqs):
    return pl.pallas_call(
        _vq_body,
        grid=(_NB,),
        in_specs=[
            _row_spec(_E_DIM), _row_spec(1), _row_spec(1),
            _full_spec((1, _NE * _K)), _full_spec((_NE * _K, _E_DIM)),
        ] + [_row_spec(_QW)] * len(xqs),
        out_specs=[_row_spec(1)],
        out_shape=[jax.ShapeDtypeStruct((_B, 1), jnp.int32)],
    )(z, lab2, rn, cn, cbl, *xqs)[0]


@jax.jit
def _run(x, labels, We0, be0, We1, be1, We2, be2,
         Wd0, bd0, Wd1, bd1, Wd2, bd2, codebooks):
    lab2 = labels.astype(jnp.int32).reshape(_B, 1)
    cbf = codebooks.reshape(_L, _NE * _K, _E_DIM)
    cbp = jnp.pad(cbf, ((0, 0), (0, 0), (0, _QW - _E_DIM)))
    biases = [b.reshape(1, -1) for b in (be0, be1, be2, bd0, bd1, bd2)]
    (be0r, be1r, be2r, bd0r, bd1r, bd2r) = biases

    z = pl.pallas_call(
        _enc_body,
        grid=(_NB,),
        in_specs=[
            _row_spec(_IN_DIM),
            _full_spec((_IN_DIM, _H1)), _full_spec((1, _H1)),
            _full_spec((_H1, _H2)), _full_spec((1, _H2)),
            _full_spec((_H2, _E_DIM)), _full_spec((1, _E_DIM)),
        ],
        out_specs=[_row_spec(_E_DIM)],
        out_shape=[jax.ShapeDtypeStruct((_B, _E_DIM), jnp.float32)],
    )(x, We0, be0r, We1, be1r, We2, be2r)[0]

    fis, xqs = [], []
    r = z
    for l in range(_L):
        rn = jnp.sum(r * r, axis=1)[:, None]
        cn = jnp.sum(cbf[l] * cbf[l], axis=1)[None, :]
        fis.append(_vq_call(z, lab2, rn, cn, cbf[l], xqs))
        xqs.append(_sc_gather(cbp[l], fis[-1].reshape(_B // _CH, _CH)))
        if l + 1 < _L:
            q = xqs[-1][:, :_E_DIM]
            u = r + (q - r)
            r = r - u

    out, xhat, l0, l1, l2 = pl.pallas_call(
        _dec_body,
        grid=(_NB,),
        in_specs=[
            _row_spec(_E_DIM), _row_spec(_QW),
            _row_spec(_QW), _row_spec(_QW),
            _full_spec((_E_DIM, _H2)), _full_spec((1, _H2)),
            _full_spec((_H2, _H1)), _full_spec((1, _H1)),
            _full_spec((_H1, _IN_DIM)), _full_spec((1, _IN_DIM)),
        ],
        out_specs=[
            _row_spec(_IN_DIM), _row_spec(_E_DIM),
            _full_spec((1, 1)), _full_spec((1, 1)), _full_spec((1, 1)),
        ],
        out_shape=[
            jax.ShapeDtypeStruct((_B, _IN_DIM), jnp.float32),
            jax.ShapeDtypeStruct((_B, _E_DIM), jnp.float32),
            jax.ShapeDtypeStruct((1, 1), jnp.float32),
            jax.ShapeDtypeStruct((1, 1), jnp.float32),
            jax.ShapeDtypeStruct((1, 1), jnp.float32),
        ],
    )(z, xqs[0], xqs[1], xqs[2], Wd0, bd0r, Wd1, bd1r, Wd2, bd2r)

    indices = jnp.concatenate(fis, axis=1) - lab2 * _K
    denom = jnp.float32(_B * _E_DIM)
    per_level = jnp.stack([l0[0, 0], l1[0, 0], l2[0, 0]]) / denom
    rq_loss = jnp.mean(per_level * (1.0 + _BETA))
    return out, rq_loss, indices, xhat


def kernel(x, labels, We0, be0, We1, be1, We2, be2,
           Wd0, bd0, Wd1, bd1, Wd2, bd2, codebooks):
    return _run(x, labels, We0, be0, We1, be1, We2, be2,
                Wd0, bd0, Wd1, bd1, Wd2, bd2, codebooks)


# speedup vs baseline: 1.1053x; 1.1053x over previous
import functools

import jax
import jax.numpy as jnp
from jax import lax
from jax.experimental import pallas as pl
from jax.experimental.pallas import tpu as pltpu
from jax.experimental.pallas import tpu_sc as plsc

_B = 16384
_IN_DIM = 768
_H1 = 512
_H2 = 256
_E_DIM = 64
_L = 3
_K = 256
_NE = 10
_BETA = 0.001
_BM = 512
_NB = _B // _BM

_SC_CORES = 2
_SC_SUBCORES = 16
_NW = _SC_CORES * _SC_SUBCORES
_BW = _B // _NW
_QW = 2 * _E_DIM
_CH = 128
_NCH = _BW // _CH


def _row_spec(w):
    return pl.BlockSpec((_BM, w), lambda i: (i, 0))


def _full_spec(shape):
    nd = len(shape)
    return pl.BlockSpec(shape, lambda i: (0,) * nd)


def _st_update(r, xhat, q):
    u = r + (q - r)
    return r - u, xhat + u


def _vq_argmin(r, cbl, cn, lab):
    rn = jnp.sum(r * r, axis=1)[:, None]
    sc = jnp.dot(r, cbl.T, preferred_element_type=jnp.float32)
    d = (rn - 2.0 * sc) + cn
    col = jax.lax.broadcasted_iota(jnp.int32, (1, _NE * _K), 1)
    d = jnp.where((col // _K) == lab, d, jnp.float32(1e30))
    return jnp.argmin(d, axis=1).astype(jnp.int32)[:, None]


def _enc_vq0_body(x_ref, lab_ref, We0, be0, We1, be1, We2, be2, cb_ref,
                  cn_ref, z_ref, fi_ref):
    h = jnp.maximum(jnp.dot(x_ref[...], We0[...],
                            preferred_element_type=jnp.float32) + be0[...], 0.0)
    h = jnp.maximum(jnp.dot(h, We1[...],
                            preferred_element_type=jnp.float32) + be1[...], 0.0)
    z = jnp.dot(h, We2[...], preferred_element_type=jnp.float32) + be2[...]
    z_ref[...] = z
    fi_ref[...] = _vq_argmin(z, cb_ref[...], cn_ref[...], lab_ref[...])


def _vq_body(*refs):
    (z_ref, lab_ref, cn_ref, cb_ref), rest = refs[:4], refs[4:]
    xq_refs, fi_ref = rest[:-1], rest[-1]
    r = z_ref[...]
    xhat = jnp.zeros_like(r)
    for q_ref in xq_refs:
        r, xhat = _st_update(r, xhat, q_ref[...][:, :_E_DIM])
    fi_ref[...] = _vq_argmin(r, cb_ref[...], cn_ref[...], lab_ref[...])


def _dec_body(z_ref, xq0_ref, xq1_ref, xq2_ref,
              Wd0, bd0, Wd1, bd1, Wd2, bd2,
              out_ref, xhat_ref, l0_ref, l1_ref, l2_ref):
    i = pl.program_id(0)
    r = z_ref[...]
    xhat = jnp.zeros_like(r)
    loss_refs = (l0_ref, l1_ref, l2_ref)
    for l, q_ref in enumerate((xq0_ref, xq1_ref, xq2_ref)):
        q = q_ref[...][:, :_E_DIM]
        diff = q - r
        lsum = jnp.sum(diff * diff).reshape(1, 1)
        r, xhat = _st_update(r, xhat, q)

        @pl.when(i == 0)
        def _():
            loss_refs[l][...] = lsum

        @pl.when(i != 0)
        def _():
            loss_refs[l][...] = loss_refs[l][...] + lsum

    xhat_ref[...] = xhat
    h = jnp.maximum(jnp.dot(xhat, Wd0[...],
                            preferred_element_type=jnp.float32) + bd0[...], 0.0)
    h = jnp.maximum(jnp.dot(h, Wd1[...],
                            preferred_element_type=jnp.float32) + bd1[...], 0.0)
    out_ref[...] = jnp.dot(h, Wd2[...],
                           preferred_element_type=jnp.float32) + bd2[...]


def _sc_gather(table, idx2d):
    mesh = plsc.VectorSubcoreMesh(core_axis_name="c", subcore_axis_name="s")

    @functools.partial(
        pl.kernel, mesh=mesh,
        out_type=jax.ShapeDtypeStruct((_B, _QW), jnp.float32),
        scratch_types=[
            pltpu.VMEM((_NCH, _CH), jnp.int32),
            pltpu.VMEM((_BW, _QW), jnp.float32),
            pltpu.SemaphoreType.DMA,
        ],
    )
    def k(table_hbm, idx_hbm, out_hbm, idx_v, rows_v, sem):
        wid = lax.axis_index("s") * _SC_CORES + lax.axis_index("c")
        pltpu.sync_copy(idx_hbm.at[pl.ds(wid * _NCH, _NCH)], idx_v)
        copies = [
            pltpu.async_copy(table_hbm.at[idx_v.at[j]],
                             rows_v.at[pl.ds(j * _CH, _CH)], sem)
            for j in range(_NCH)
        ]
        for c in copies:
            c.wait()
        pltpu.sync_copy(rows_v, out_hbm.at[pl.ds(wid * _BW, _BW)])

    return k(table, idx2d)


def _vq_call(z, lab2, cn, cbl, xqs):
    return pl.pallas_call(
        _vq_body,
        grid=(_NB,),
        in_specs=[
            _row_spec(_E_DIM), _row_spec(1),
            _full_spec((1, _NE * _K)), _full_spec((_NE * _K, _E_DIM)),
        ] + [_row_spec(_QW)] * len(xqs),
        out_specs=[_row_spec(1)],
        out_shape=[jax.ShapeDtypeStruct((_B, 1), jnp.int32)],
    )(z, lab2, cn, cbl, *xqs)[0]


@jax.jit
def _run(x, labels, We0, be0, We1, be1, We2, be2,
         Wd0, bd0, Wd1, bd1, Wd2, bd2, codebooks):
    lab2 = labels.astype(jnp.int32).reshape(_B, 1)
    cbf = codebooks.reshape(_L, _NE * _K, _E_DIM)
    cbp = jnp.pad(cbf, ((0, 0), (0, 0), (0, _QW - _E_DIM)))
    biases = [b.reshape(1, -1) for b in (be0, be1, be2, bd0, bd1, bd2)]
    (be0r, be1r, be2r, bd0r, bd1r, bd2r) = biases

    cns = [jnp.sum(cbf[l] * cbf[l], axis=1)[None, :] for l in range(_L)]

    z, fi0 = pl.pallas_call(
        _enc_vq0_body,
        grid=(_NB,),
        in_specs=[
            _row_spec(_IN_DIM), _row_spec(1),
            _full_spec((_IN_DIM, _H1)), _full_spec((1, _H1)),
            _full_spec((_H1, _H2)), _full_spec((1, _H2)),
            _full_spec((_H2, _E_DIM)), _full_spec((1, _E_DIM)),
            _full_spec((_NE * _K, _E_DIM)), _full_spec((1, _NE * _K)),
        ],
        out_specs=[_row_spec(_E_DIM), _row_spec(1)],
        out_shape=[
            jax.ShapeDtypeStruct((_B, _E_DIM), jnp.float32),
            jax.ShapeDtypeStruct((_B, 1), jnp.int32),
        ],
    )(x, lab2, We0, be0r, We1, be1r, We2, be2r, cbf[0], cns[0])

    fis, xqs = [fi0], []
    for l in range(_L):
        if l > 0:
            fis.append(_vq_call(z, lab2, cns[l], cbf[l], xqs))
        xqs.append(_sc_gather(cbp[l], fis[-1].reshape(_B // _CH, _CH)))

    out, xhat, l0, l1, l2 = pl.pallas_call(
        _dec_body,
        grid=(_NB,),
        in_specs=[
            _row_spec(_E_DIM), _row_spec(_QW),
            _row_spec(_QW), _row_spec(_QW),
            _full_spec((_E_DIM, _H2)), _full_spec((1, _H2)),
            _full_spec((_H2, _H1)), _full_spec((1, _H1)),
            _full_spec((_H1, _IN_DIM)), _full_spec((1, _IN_DIM)),
        ],
        out_specs=[
            _row_spec(_IN_DIM), _row_spec(_E_DIM),
            _full_spec((1, 1)), _full_spec((1, 1)), _full_spec((1, 1)),
        ],
        out_shape=[
            jax.ShapeDtypeStruct((_B, _IN_DIM), jnp.float32),
            jax.ShapeDtypeStruct((_B, _E_DIM), jnp.float32),
            jax.ShapeDtypeStruct((1, 1), jnp.float32),
            jax.ShapeDtypeStruct((1, 1), jnp.float32),
            jax.ShapeDtypeStruct((1, 1), jnp.float32),
        ],
    )(z, xqs[0], xqs[1], xqs[2], Wd0, bd0r, Wd1, bd1r, Wd2, bd2r)

    indices = jnp.concatenate(fis, axis=1) - lab2 * _K
    denom = jnp.float32(_B * _E_DIM)
    per_level = jnp.stack([l0[0, 0], l1[0, 0], l2[0, 0]]) / denom
    rq_loss = jnp.mean(per_level * (1.0 + _BETA))
    return out, rq_loss, indices, xhat


def kernel(x, labels, We0, be0, We1, be1, We2, be2,
           Wd0, bd0, Wd1, bd1, Wd2, bd2, codebooks):
    return _run(x, labels, We0, be0, We1, be1, We2, be2,
                Wd0, bd0, Wd1, bd1, Wd2, bd2, codebooks)


# speedup vs baseline: 1.1905x; 1.0771x over previous
import functools

import jax
import jax.numpy as jnp
from jax import lax
from jax.experimental import pallas as pl
from jax.experimental.pallas import tpu as pltpu
from jax.experimental.pallas import tpu_sc as plsc

_B = 16384
_IN_DIM = 768
_H1 = 512
_H2 = 256
_E_DIM = 64
_L = 3
_K = 256
_NE = 10
_BETA = 0.001
_BM = 1024
_NB = _B // _BM

_SC_CORES = 2
_SC_SUBCORES = 16
_NW = _SC_CORES * _SC_SUBCORES
_BW = _B // _NW
_QW = 2 * _E_DIM
_CH = 128
_NCH = _BW // _CH


def _row_spec(w):
    return pl.BlockSpec((_BM, w), lambda i: (i, 0))


def _full_spec(shape):
    nd = len(shape)
    return pl.BlockSpec(shape, lambda i: (0,) * nd)


def _st_update(r, xhat, q):
    u = r + (q - r)
    return r - u, xhat + u


def _vq_argmin(r, cbl, cn, lab):
    rn = jnp.sum(r * r, axis=1)[:, None]
    sc = jnp.dot(r, cbl.T, preferred_element_type=jnp.float32)
    d = (rn - 2.0 * sc) + cn
    col = jax.lax.broadcasted_iota(jnp.int32, (1, _NE * _K), 1)
    d = jnp.where((col // _K) == lab, d, jnp.float32(1e30))
    return jnp.argmin(d, axis=1).astype(jnp.int32)[:, None]


def _enc_vq0_body(x_ref, lab_ref, We0, be0, We1, be1, We2, be2, cb_ref,
                  cn_ref, z_ref, fi_ref):
    h = jnp.maximum(jnp.dot(x_ref[...], We0[...],
                            preferred_element_type=jnp.float32) + be0[...], 0.0)
    h = jnp.maximum(jnp.dot(h, We1[...],
                            preferred_element_type=jnp.float32) + be1[...], 0.0)
    z = jnp.dot(h, We2[...], preferred_element_type=jnp.float32) + be2[...]
    z_ref[...] = z
    fi_ref[...] = _vq_argmin(z, cb_ref[...], cn_ref[...], lab_ref[...])


def _vq_body(*refs):
    (z_ref, lab_ref, cn_ref, cb_ref), rest = refs[:4], refs[4:]
    xq_refs, fi_ref = rest[:-1], rest[-1]
    r = z_ref[...]
    xhat = jnp.zeros_like(r)
    for q_ref in xq_refs:
        r, xhat = _st_update(r, xhat, q_ref[...][:, :_E_DIM])
    fi_ref[...] = _vq_argmin(r, cb_ref[...], cn_ref[...], lab_ref[...])


def _dec_body(z_ref, xq0_ref, xq1_ref, xq2_ref,
              Wd0, bd0, Wd1, bd1, Wd2, bd2,
              out_ref, xhat_ref, l0_ref, l1_ref, l2_ref):
    i = pl.program_id(0)
    r = z_ref[...]
    xhat = jnp.zeros_like(r)
    loss_refs = (l0_ref, l1_ref, l2_ref)
    for l, q_ref in enumerate((xq0_ref, xq1_ref, xq2_ref)):
        q = q_ref[...][:, :_E_DIM]
        diff = q - r
        lsum = jnp.sum(diff * diff).reshape(1, 1)
        r, xhat = _st_update(r, xhat, q)

        @pl.when(i == 0)
        def _():
            loss_refs[l][...] = lsum

        @pl.when(i != 0)
        def _():
            loss_refs[l][...] = loss_refs[l][...] + lsum

    xhat_ref[...] = xhat
    h = jnp.maximum(jnp.dot(xhat, Wd0[...],
                            preferred_element_type=jnp.float32) + bd0[...], 0.0)
    h = jnp.maximum(jnp.dot(h, Wd1[...],
                            preferred_element_type=jnp.float32) + bd1[...], 0.0)
    out_ref[...] = jnp.dot(h, Wd2[...],
                           preferred_element_type=jnp.float32) + bd2[...]


def _sc_gather(table, idx2d):
    mesh = plsc.VectorSubcoreMesh(core_axis_name="c", subcore_axis_name="s")

    @functools.partial(
        pl.kernel, mesh=mesh,
        out_type=jax.ShapeDtypeStruct((_B, _QW), jnp.float32),
        scratch_types=[
            pltpu.VMEM((_NCH, _CH), jnp.int32),
            pltpu.VMEM((_BW, _QW), jnp.float32),
            pltpu.SemaphoreType.DMA,
        ],
    )
    def k(table_hbm, idx_hbm, out_hbm, idx_v, rows_v, sem):
        wid = lax.axis_index("s") * _SC_CORES + lax.axis_index("c")
        pltpu.sync_copy(idx_hbm.at[pl.ds(wid * _NCH, _NCH)], idx_v)
        copies = [
            pltpu.async_copy(table_hbm.at[idx_v.at[j]],
                             rows_v.at[pl.ds(j * _CH, _CH)], sem)
            for j in range(_NCH)
        ]
        for c in copies:
            c.wait()
        pltpu.sync_copy(rows_v, out_hbm.at[pl.ds(wid * _BW, _BW)])

    return k(table, idx2d)


def _vq_call(z, lab2, cn, cbl, xqs):
    return pl.pallas_call(
        _vq_body,
        grid=(_NB,),
        in_specs=[
            _row_spec(_E_DIM), _row_spec(1),
            _full_spec((1, _NE * _K)), _full_spec((_NE * _K, _E_DIM)),
        ] + [_row_spec(_QW)] * len(xqs),
        out_specs=[_row_spec(1)],
        out_shape=[jax.ShapeDtypeStruct((_B, 1), jnp.int32)],
    )(z, lab2, cn, cbl, *xqs)[0]


@jax.jit
def _run(x, labels, We0, be0, We1, be1, We2, be2,
         Wd0, bd0, Wd1, bd1, Wd2, bd2, codebooks):
    lab2 = labels.astype(jnp.int32).reshape(_B, 1)
    cbf = codebooks.reshape(_L, _NE * _K, _E_DIM)
    cbp = jnp.pad(cbf, ((0, 0), (0, 0), (0, _QW - _E_DIM)))
    biases = [b.reshape(1, -1) for b in (be0, be1, be2, bd0, bd1, bd2)]
    (be0r, be1r, be2r, bd0r, bd1r, bd2r) = biases

    cns = [jnp.sum(cbf[l] * cbf[l], axis=1)[None, :] for l in range(_L)]

    z, fi0 = pl.pallas_call(
        _enc_vq0_body,
        grid=(_NB,),
        in_specs=[
            _row_spec(_IN_DIM), _row_spec(1),
            _full_spec((_IN_DIM, _H1)), _full_spec((1, _H1)),
            _full_spec((_H1, _H2)), _full_spec((1, _H2)),
            _full_spec((_H2, _E_DIM)), _full_spec((1, _E_DIM)),
            _full_spec((_NE * _K, _E_DIM)), _full_spec((1, _NE * _K)),
        ],
        out_specs=[_row_spec(_E_DIM), _row_spec(1)],
        out_shape=[
            jax.ShapeDtypeStruct((_B, _E_DIM), jnp.float32),
            jax.ShapeDtypeStruct((_B, 1), jnp.int32),
        ],
    )(x, lab2, We0, be0r, We1, be1r, We2, be2r, cbf[0], cns[0])

    fis, xqs = [fi0], []
    for l in range(_L):
        if l > 0:
            fis.append(_vq_call(z, lab2, cns[l], cbf[l], xqs))
        xqs.append(_sc_gather(cbp[l], fis[-1].reshape(_B // _CH, _CH)))

    out, xhat, l0, l1, l2 = pl.pallas_call(
        _dec_body,
        grid=(_NB,),
        in_specs=[
            _row_spec(_E_DIM), _row_spec(_QW),
            _row_spec(_QW), _row_spec(_QW),
            _full_spec((_E_DIM, _H2)), _full_spec((1, _H2)),
            _full_spec((_H2, _H1)), _full_spec((1, _H1)),
            _full_spec((_H1, _IN_DIM)), _full_spec((1, _IN_DIM)),
        ],
        out_specs=[
            _row_spec(_IN_DIM), _row_spec(_E_DIM),
            _full_spec((1, 1)), _full_spec((1, 1)), _full_spec((1, 1)),
        ],
        out_shape=[
            jax.ShapeDtypeStruct((_B, _IN_DIM), jnp.float32),
            jax.ShapeDtypeStruct((_B, _E_DIM), jnp.float32),
            jax.ShapeDtypeStruct((1, 1), jnp.float32),
            jax.ShapeDtypeStruct((1, 1), jnp.float32),
            jax.ShapeDtypeStruct((1, 1), jnp.float32),
        ],
    )(z, xqs[0], xqs[1], xqs[2], Wd0, bd0r, Wd1, bd1r, Wd2, bd2r)

    indices = jnp.concatenate(fis, axis=1) - lab2 * _K
    denom = jnp.float32(_B * _E_DIM)
    per_level = jnp.stack([l0[0, 0], l1[0, 0], l2[0, 0]]) / denom
    rq_loss = jnp.mean(per_level * (1.0 + _BETA))
    return out, rq_loss, indices, xhat


def kernel(x, labels, We0, be0, We1, be1, We2, be2,
           Wd0, bd0, Wd1, bd1, Wd2, bd2, codebooks):
    return _run(x, labels, We0, be0, We1, be1, We2, be2,
                Wd0, bd0, Wd1, bd1, Wd2, bd2, codebooks)


# speedup vs baseline: 1.2164x; 1.0218x over previous
import functools

import jax
import jax.numpy as jnp
from jax import lax
from jax.experimental import pallas as pl
from jax.experimental.pallas import tpu as pltpu
from jax.experimental.pallas import tpu_sc as plsc

_B = 16384
_IN_DIM = 768
_H1 = 512
_H2 = 256
_E_DIM = 64
_L = 3
_K = 256
_NE = 10
_BETA = 0.001
_BM = 2048
_NB = _B // _BM

_SC_CORES = 2
_SC_SUBCORES = 16
_NW = _SC_CORES * _SC_SUBCORES
_BW = _B // _NW
_QW = 2 * _E_DIM
_CH = 128
_NCH = _BW // _CH


def _row_spec(w):
    return pl.BlockSpec((_BM, w), lambda i: (i, 0))


def _full_spec(shape):
    nd = len(shape)
    return pl.BlockSpec(shape, lambda i: (0,) * nd)


def _st_update(r, xhat, q):
    u = r + (q - r)
    return r - u, xhat + u


def _vq_argmin(r, cbl, cn, lab):
    rn = jnp.sum(r * r, axis=1)[:, None]
    sc = jnp.dot(r, cbl.T, preferred_element_type=jnp.float32)
    d = (rn - 2.0 * sc) + cn
    col = jax.lax.broadcasted_iota(jnp.int32, (1, _NE * _K), 1)
    d = jnp.where((col // _K) == lab, d, jnp.float32(1e30))
    return jnp.argmin(d, axis=1).astype(jnp.int32)[:, None]


def _enc_vq0_body(x_ref, lab_ref, We0, be0, We1, be1, We2, be2, cb_ref,
                  cn_ref, z_ref, fi_ref):
    h = jnp.maximum(jnp.dot(x_ref[...], We0[...],
                            preferred_element_type=jnp.float32) + be0[...], 0.0)
    h = jnp.maximum(jnp.dot(h, We1[...],
                            preferred_element_type=jnp.float32) + be1[...], 0.0)
    z = jnp.dot(h, We2[...], preferred_element_type=jnp.float32) + be2[...]
    z_ref[...] = z
    fi_ref[...] = _vq_argmin(z, cb_ref[...], cn_ref[...], lab_ref[...])


def _vq_body(*refs):
    (z_ref, lab_ref, cn_ref, cb_ref), rest = refs[:4], refs[4:]
    xq_refs, fi_ref = rest[:-1], rest[-1]
    r = z_ref[...]
    xhat = jnp.zeros_like(r)
    for q_ref in xq_refs:
        r, xhat = _st_update(r, xhat, q_ref[...][:, :_E_DIM])
    fi_ref[...] = _vq_argmin(r, cb_ref[...], cn_ref[...], lab_ref[...])


def _dec_body(z_ref, xq0_ref, xq1_ref, xq2_ref,
              Wd0, bd0, Wd1, bd1, Wd2, bd2,
              out_ref, xhat_ref, l0_ref, l1_ref, l2_ref):
    i = pl.program_id(0)
    r = z_ref[...]
    xhat = jnp.zeros_like(r)
    loss_refs = (l0_ref, l1_ref, l2_ref)
    for l, q_ref in enumerate((xq0_ref, xq1_ref, xq2_ref)):
        q = q_ref[...][:, :_E_DIM]
        diff = q - r
        lsum = jnp.sum(diff * diff).reshape(1, 1)
        r, xhat = _st_update(r, xhat, q)

        @pl.when(i == 0)
        def _():
            loss_refs[l][...] = lsum

        @pl.when(i != 0)
        def _():
            loss_refs[l][...] = loss_refs[l][...] + lsum

    xhat_ref[...] = xhat
    h = jnp.maximum(jnp.dot(xhat, Wd0[...],
                            preferred_element_type=jnp.float32) + bd0[...], 0.0)
    h = jnp.maximum(jnp.dot(h, Wd1[...],
                            preferred_element_type=jnp.float32) + bd1[...], 0.0)
    out_ref[...] = jnp.dot(h, Wd2[...],
                           preferred_element_type=jnp.float32) + bd2[...]


def _sc_gather(table, idx2d):
    mesh = plsc.VectorSubcoreMesh(core_axis_name="c", subcore_axis_name="s")

    @functools.partial(
        pl.kernel, mesh=mesh,
        out_type=jax.ShapeDtypeStruct((_B, _QW), jnp.float32),
        scratch_types=[
            pltpu.VMEM((_NCH, _CH), jnp.int32),
            pltpu.VMEM((_BW, _QW), jnp.float32),
            pltpu.SemaphoreType.DMA,
        ],
    )
    def k(table_hbm, idx_hbm, out_hbm, idx_v, rows_v, sem):
        wid = lax.axis_index("s") * _SC_CORES + lax.axis_index("c")
        pltpu.sync_copy(idx_hbm.at[pl.ds(wid * _NCH, _NCH)], idx_v)
        copies = [
            pltpu.async_copy(table_hbm.at[idx_v.at[j]],
                             rows_v.at[pl.ds(j * _CH, _CH)], sem)
            for j in range(_NCH)
        ]
        for c in copies:
            c.wait()
        pltpu.sync_copy(rows_v, out_hbm.at[pl.ds(wid * _BW, _BW)])

    return k(table, idx2d)


def _vq_call(z, lab2, cn, cbl, xqs):
    return pl.pallas_call(
        _vq_body,
        grid=(_NB,),
        in_specs=[
            _row_spec(_E_DIM), _row_spec(1),
            _full_spec((1, _NE * _K)), _full_spec((_NE * _K, _E_DIM)),
        ] + [_row_spec(_QW)] * len(xqs),
        out_specs=[_row_spec(1)],
        out_shape=[jax.ShapeDtypeStruct((_B, 1), jnp.int32)],
    )(z, lab2, cn, cbl, *xqs)[0]


@jax.jit
def _run(x, labels, We0, be0, We1, be1, We2, be2,
         Wd0, bd0, Wd1, bd1, Wd2, bd2, codebooks):
    lab2 = labels.astype(jnp.int32).reshape(_B, 1)
    cbf = codebooks.reshape(_L, _NE * _K, _E_DIM)
    cbp = jnp.pad(cbf, ((0, 0), (0, 0), (0, _QW - _E_DIM)))
    biases = [b.reshape(1, -1) for b in (be0, be1, be2, bd0, bd1, bd2)]
    (be0r, be1r, be2r, bd0r, bd1r, bd2r) = biases

    cns = [jnp.sum(cbf[l] * cbf[l], axis=1)[None, :] for l in range(_L)]

    z, fi0 = pl.pallas_call(
        _enc_vq0_body,
        grid=(_NB,),
        in_specs=[
            _row_spec(_IN_DIM), _row_spec(1),
            _full_spec((_IN_DIM, _H1)), _full_spec((1, _H1)),
            _full_spec((_H1, _H2)), _full_spec((1, _H2)),
            _full_spec((_H2, _E_DIM)), _full_spec((1, _E_DIM)),
            _full_spec((_NE * _K, _E_DIM)), _full_spec((1, _NE * _K)),
        ],
        out_specs=[_row_spec(_E_DIM), _row_spec(1)],
        out_shape=[
            jax.ShapeDtypeStruct((_B, _E_DIM), jnp.float32),
            jax.ShapeDtypeStruct((_B, 1), jnp.int32),
        ],
    )(x, lab2, We0, be0r, We1, be1r, We2, be2r, cbf[0], cns[0])

    fis, xqs = [fi0], []
    for l in range(_L):
        if l > 0:
            fis.append(_vq_call(z, lab2, cns[l], cbf[l], xqs))
        xqs.append(_sc_gather(cbp[l], fis[-1].reshape(_B // _CH, _CH)))

    out, xhat, l0, l1, l2 = pl.pallas_call(
        _dec_body,
        grid=(_NB,),
        in_specs=[
            _row_spec(_E_DIM), _row_spec(_QW),
            _row_spec(_QW), _row_spec(_QW),
            _full_spec((_E_DIM, _H2)), _full_spec((1, _H2)),
            _full_spec((_H2, _H1)), _full_spec((1, _H1)),
            _full_spec((_H1, _IN_DIM)), _full_spec((1, _IN_DIM)),
        ],
        out_specs=[
            _row_spec(_IN_DIM), _row_spec(_E_DIM),
            _full_spec((1, 1)), _full_spec((1, 1)), _full_spec((1, 1)),
        ],
        out_shape=[
            jax.ShapeDtypeStruct((_B, _IN_DIM), jnp.float32),
            jax.ShapeDtypeStruct((_B, _E_DIM), jnp.float32),
            jax.ShapeDtypeStruct((1, 1), jnp.float32),
            jax.ShapeDtypeStruct((1, 1), jnp.float32),
            jax.ShapeDtypeStruct((1, 1), jnp.float32),
        ],
    )(z, xqs[0], xqs[1], xqs[2], Wd0, bd0r, Wd1, bd1r, Wd2, bd2r)

    indices = jnp.concatenate(fis, axis=1) - lab2 * _K
    denom = jnp.float32(_B * _E_DIM)
    per_level = jnp.stack([l0[0, 0], l1[0, 0], l2[0, 0]]) / denom
    rq_loss = jnp.mean(per_level * (1.0 + _BETA))
    return out, rq_loss, indices, xhat


def kernel(x, labels, We0, be0, We1, be1, We2, be2,
           Wd0, bd0, Wd1, bd1, Wd2, bd2, codebooks):
    return _run(x, labels, We0, be0, We1, be1, We2, be2,
                Wd0, bd0, Wd1, bd1, Wd2, bd2, codebooks)
